# baseline (speedup 1.0000x reference)
import functools

import jax
import jax.numpy as jnp
from jax import lax
from jax.experimental import pallas as pl
from jax.experimental.pallas import tpu as pltpu
from jax.experimental.pallas import tpu_sc as plsc

N = 10000
E = 320000
IN_F = 128
F = 64
D = 16

NC = 2
NS = 16
NW = NC * NS
EW = E // NW
K = 16
QUAD = 5
NQUAD = EW // (QUAD * K)


def _embed_body(x_ref, w_ref, b_ref, h_ref):
    h_ref[...] = (
        jnp.dot(x_ref[...], w_ref[...], preferred_element_type=jnp.float32)
        + b_ref[...]
    )


def _embed(x, w, b):
    return pl.pallas_call(
        _embed_body,
        out_shape=jax.ShapeDtypeStruct((N, F), jnp.float32),
    )(x, w, b)


def _tables_body(h_ref, wd_ref, ws_ref, td_ref, ts_ref):
    h = h_ref[...]
    td_ref[...] = jnp.dot(h, wd_ref[...], preferred_element_type=jnp.float32)
    ts_ref[...] = jnp.dot(h, ws_ref[...], preferred_element_type=jnp.float32)


def _tables(h, wd, wsrc):
    return pl.pallas_call(
        _tables_body,
        out_shape=(
            jax.ShapeDtypeStruct((N, 2 * F), jnp.float32),
            jax.ShapeDtypeStruct((N, 2 * F), jnp.float32),
        ),
    )(h, wd, wsrc)


_EB = 16000


def _cmat_body(ea_ref, wc_ref, bc_ref, c_ref):
    c_ref[...] = (
        jnp.dot(ea_ref[...], wc_ref[...], preferred_element_type=jnp.float32)
        + bc_ref[...]
    )


def _cmat(ea, wc, bc):
    return pl.pallas_call(
        _cmat_body,
        grid=(E // _EB,),
        in_specs=[
            pl.BlockSpec((_EB, D), lambda i: (i, 0)),
            pl.BlockSpec((D, 2 * F), lambda i: (0, 0)),
            pl.BlockSpec((1, 2 * F), lambda i: (0, 0)),
        ],
        out_specs=pl.BlockSpec((_EB, 2 * F), lambda i: (i, 0)),
        out_shape=jax.ShapeDtypeStruct((E, 2 * F), jnp.float32),
    )(ea, wc, bc)


def _bn_body(p_ref, h_ref, g_ref, be_ref, o_ref):
    agg = p_ref[0, :, :F] + p_ref[1, :, :F]
    mu = jnp.mean(agg, axis=0, keepdims=True)
    cen = agg - mu
    var = jnp.mean(cen * cen, axis=0, keepdims=True)
    o_ref[...] = g_ref[...] * cen * lax.rsqrt(var + 1e-5) + be_ref[...] + h_ref[...]


def _bn(p, h, g, be):
    return pl.pallas_call(
        _bn_body,
        out_shape=jax.ShapeDtypeStruct((N, F), jnp.float32),
    )(p, h, g, be)


def _softplus16(z):
    u = jnp.exp(-jnp.abs(z))
    t = u / (2.0 + u)
    t2 = t * t
    y0 = t * (2.0 + t2 * (2.0 / 3.0 + t2 * 0.4))
    return jnp.maximum(z, 0.0) + y0


def _sc_body(td_hbm, ts_hbm, c_hbm, src4_hbm, dst4_hbm, zero_hbm, out_hbm,
             idx_s4, idx_d4, *rest):
    bufs = rest[:4 * QUAD]
    agg = rest[4 * QUAD]
    sems = rest[4 * QUAD + 1:]
    cid = lax.axis_index("c")
    sid = lax.axis_index("s")
    wid = sid * NC + cid
    wbase = pl.multiple_of(wid * EW, 8)

    @pl.when(sid == 0)
    def _():
        pltpu.sync_copy(zero_hbm, agg)

    for i in range(QUAD):
        mb = bufs[4 * i + 3]

        @plsc.parallel_loop(0, K)
        def _(e):
            for j in range(F // 16):
                mb[e, pl.ds(F + 16 * j, 16)] = jnp.zeros((16,), jnp.float32)

    plsc.subcore_barrier()

    def compute(bd, bs, bc, mb):
        @plsc.parallel_loop(0, K, unroll=2)
        def _(e):
            for j in range(F // 16):
                f_sl = pl.ds(16 * j, 16)
                s_sl = pl.ds(F + 16 * j, 16)
                zf = bd[e, f_sl] + bs[e, f_sl] + bc[e, f_sl]
                zs = bd[e, s_sl] + bs[e, s_sl] + bc[e, s_sl]
                mb[e, f_sl] = _softplus16(zs) / (1.0 + jnp.exp(-zf))

    def quad(t, carry):
        pr = wid * NQUAD + t
        pltpu.sync_copy(src4_hbm.at[pr], idx_s4)
        pltpu.sync_copy(dst4_hbm.at[pr], idx_d4)
        gh = []
        for i in range(QUAD):
            bd, bs, bc, _ = bufs[4 * i:4 * i + 4]
            semd, semsrc, semc, _ = sems[4 * i:4 * i + 4]
            gh.append((
                pltpu.async_copy(td_hbm.at[idx_d4.at[i]], bd, semd),
                pltpu.async_copy(ts_hbm.at[idx_s4.at[i]], bs, semsrc),
                pltpu.async_copy(
                    c_hbm.at[pl.ds(wbase + (QUAD * t + i) * K, K), :], bc, semc),
            ))
        sh = []
        for i in range(QUAD):
            bd, bs, bc, mb = bufs[4 * i:4 * i + 4]
            for h in gh[i]:
                h.wait()
            compute(bd, bs, bc, mb)
            sh.append(pltpu.async_copy(
                mb, agg.at[idx_d4.at[i]], sems[4 * i + 3], add=True))
        for h in sh:
            h.wait()
        return carry

    lax.fori_loop(0, NQUAD, quad, 0)

    plsc.subcore_barrier()

    rows = 624
    sl = pl.ds(sid * rows, rows)
    pltpu.sync_copy(agg.at[sl, :], out_hbm.at[cid, sl, :])

    @pl.when(sid == 0)
    def _():
        tail = pl.ds(NS * rows, N - NS * rows)
        pltpu.sync_copy(agg.at[tail, :], out_hbm.at[cid, tail, :])


@functools.cache
def _get_sc_layer():
    return functools.partial(
        pl.kernel,
        mesh=plsc.VectorSubcoreMesh(core_axis_name="c", subcore_axis_name="s"),
        out_type=jax.ShapeDtypeStruct((NC, N, 2 * F), jnp.float32),
        scratch_types=(
            [
                pltpu.VMEM((QUAD, K), jnp.int32),
                pltpu.VMEM((QUAD, K), jnp.int32),
            ]
            + [pltpu.VMEM((K, 2 * F), jnp.float32)] * (4 * QUAD)
            + [pltpu.VMEM_SHARED((N, 2 * F), jnp.float32)]
            + [pltpu.SemaphoreType.DMA] * (4 * QUAD)
        ),
    )(_sc_body)


def _sc_layer(td, ts, c, src, dst, zero):
    return _get_sc_layer()(td, ts, c, src, dst, zero)


def kernel(x, edge_index, edge_attr, batch, W_emb, b_emb,
           Wf0, bf0, Ws0, bs0, g0, be0,
           Wf1, bf1, Ws1, bs1, g1, be1,
           Wf2, bf2, Ws2, bs2, g2, be2):
    src4 = edge_index[0].reshape(E // (QUAD * K), QUAD, K)
    dst4 = edge_index[1].reshape(E // (QUAD * K), QUAD, K)
    zero = jnp.zeros((N, 2 * F), jnp.float32)

    h = _embed(x, W_emb, b_emb.reshape(1, F))

    for Wf, bf, Ws, bs, g, be in (
        (Wf0, bf0, Ws0, bs0, g0, be0),
        (Wf1, bf1, Ws1, bs1, g1, be1),
        (Wf2, bf2, Ws2, bs2, g2, be2),
    ):
        wd = jnp.concatenate([Wf[:F], Ws[:F]], axis=1)
        wsrc = jnp.concatenate([Wf[F:2 * F], Ws[F:2 * F]], axis=1)
        wc = jnp.concatenate([Wf[2 * F:], Ws[2 * F:]], axis=1)
        bc = jnp.concatenate([bf, bs]).reshape(1, 2 * F)
        c = _cmat(edge_attr, wc, bc)
        td, ts = _tables(h, wd, wsrc)
        p = _sc_layer(td, ts, c, src4, dst4, zero)
        h = _bn(p, h, g.reshape(1, F), be.reshape(1, F))
    return h


# speedup vs baseline: 1.4632x; 1.4632x over previous
import functools

import jax
import jax.numpy as jnp
from jax import lax
from jax.experimental import pallas as pl
from jax.experimental.pallas import tpu as pltpu
from jax.experimental.pallas import tpu_sc as plsc

N = 10000
E = 320000
IN_F = 128
F = 64
D = 16

NC = 2
NS = 16
NW = NC * NS
EW = E // NW
K = 40
QUAD = 2
NQUAD = EW // (QUAD * K)


def _embed_body(x_ref, w_ref, b_ref, h_ref):
    h_ref[...] = (
        jnp.dot(x_ref[...], w_ref[...], preferred_element_type=jnp.float32)
        + b_ref[...]
    )


def _embed(x, w, b):
    return pl.pallas_call(
        _embed_body,
        out_shape=jax.ShapeDtypeStruct((N, F), jnp.float32),
    )(x, w, b)


def _tables_body(h_ref, wd_ref, ws_ref, td_ref, ts_ref):
    h = h_ref[...]
    td_ref[...] = jnp.dot(h, wd_ref[...], preferred_element_type=jnp.float32)
    ts_ref[...] = jnp.dot(h, ws_ref[...], preferred_element_type=jnp.float32)


def _tables(h, wd, wsrc):
    return pl.pallas_call(
        _tables_body,
        out_shape=(
            jax.ShapeDtypeStruct((N, 2 * F), jnp.float32),
            jax.ShapeDtypeStruct((N, 2 * F), jnp.float32),
        ),
    )(h, wd, wsrc)


_EB = 16000


def _cmat_body(ea_ref, wc_ref, bc_ref, c_ref):
    c_ref[...] = (
        jnp.dot(ea_ref[...], wc_ref[...], preferred_element_type=jnp.float32)
        + bc_ref[...]
    )


def _cmat(ea, wc, bc):
    return pl.pallas_call(
        _cmat_body,
        grid=(E // _EB,),
        in_specs=[
            pl.BlockSpec((_EB, D), lambda i: (i, 0)),
            pl.BlockSpec((D, 2 * F), lambda i: (0, 0)),
            pl.BlockSpec((1, 2 * F), lambda i: (0, 0)),
        ],
        out_specs=pl.BlockSpec((_EB, 2 * F), lambda i: (i, 0)),
        out_shape=jax.ShapeDtypeStruct((E, 2 * F), jnp.float32),
    )(ea, wc, bc)


def _bn_body(p_ref, h_ref, g_ref, be_ref, o_ref):
    agg = p_ref[0, :, :F] + p_ref[1, :, :F]
    mu = jnp.mean(agg, axis=0, keepdims=True)
    cen = agg - mu
    var = jnp.mean(cen * cen, axis=0, keepdims=True)
    o_ref[...] = g_ref[...] * cen * lax.rsqrt(var + 1e-5) + be_ref[...] + h_ref[...]


def _bn(p, h, g, be):
    return pl.pallas_call(
        _bn_body,
        out_shape=jax.ShapeDtypeStruct((N, F), jnp.float32),
    )(p, h, g, be)


def _softplus16(z):
    u = jnp.exp(-jnp.abs(z))
    t = u / (2.0 + u)
    t2 = t * t
    y0 = t * (2.0 + t2 * (2.0 / 3.0 + t2 * 0.4))
    return jnp.maximum(z, 0.0) + y0


def _sc_body(td_hbm, ts_hbm, c_hbm, src4_hbm, dst4_hbm, zero_hbm, out_hbm,
             idx_s4, idx_d4, *rest):
    bufs = rest[:4 * QUAD]
    agg = rest[4 * QUAD]
    sems = rest[4 * QUAD + 1:]
    cid = lax.axis_index("c")
    sid = lax.axis_index("s")
    wid = sid * NC + cid
    wbase = pl.multiple_of(wid * EW, 8)

    @pl.when(sid == 0)
    def _():
        pltpu.sync_copy(zero_hbm, agg)

    for i in range(QUAD):
        mb = bufs[4 * i + 3]

        @plsc.parallel_loop(0, K)
        def _(e):
            for j in range(F // 16):
                mb[e, pl.ds(F + 16 * j, 16)] = jnp.zeros((16,), jnp.float32)

    plsc.subcore_barrier()

    def compute(bd, bs, bc, mb):
        @plsc.parallel_loop(0, K, unroll=2)
        def _(e):
            for j in range(F // 16):
                f_sl = pl.ds(16 * j, 16)
                s_sl = pl.ds(F + 16 * j, 16)
                zf = bd[e, f_sl] + bs[e, f_sl] + bc[e, f_sl]
                zs = bd[e, s_sl] + bs[e, s_sl] + bc[e, s_sl]
                mb[e, f_sl] = _softplus16(zs) / (1.0 + jnp.exp(-zf))

    def quad(t, carry):
        pr = wid * NQUAD + t
        pltpu.sync_copy(src4_hbm.at[pr], idx_s4)
        pltpu.sync_copy(dst4_hbm.at[pr], idx_d4)
        gh = []
        for i in range(QUAD):
            bd, bs, bc, _ = bufs[4 * i:4 * i + 4]
            semd, semsrc, semc, _ = sems[4 * i:4 * i + 4]
            gh.append((
                pltpu.async_copy(td_hbm.at[idx_d4.at[i]], bd, semd),
                pltpu.async_copy(ts_hbm.at[idx_s4.at[i]], bs, semsrc),
                pltpu.async_copy(
                    c_hbm.at[pl.ds(wbase + (QUAD * t + i) * K, K), :], bc, semc),
            ))
        sh = []
        for i in range(QUAD):
            bd, bs, bc, mb = bufs[4 * i:4 * i + 4]
            for h in gh[i]:
                h.wait()
            compute(bd, bs, bc, mb)
            sh.append(pltpu.async_copy(
                mb, agg.at[idx_d4.at[i]], sems[4 * i + 3], add=True))
        for h in sh:
            h.wait()
        return carry

    lax.fori_loop(0, NQUAD, quad, 0)

    plsc.subcore_barrier()

    rows = 624
    sl = pl.ds(sid * rows, rows)
    pltpu.sync_copy(agg.at[sl, :], out_hbm.at[cid, sl, :])

    @pl.when(sid == 0)
    def _():
        tail = pl.ds(NS * rows, N - NS * rows)
        pltpu.sync_copy(agg.at[tail, :], out_hbm.at[cid, tail, :])


@functools.cache
def _get_sc_layer():
    return functools.partial(
        pl.kernel,
        mesh=plsc.VectorSubcoreMesh(core_axis_name="c", subcore_axis_name="s"),
        out_type=jax.ShapeDtypeStruct((NC, N, 2 * F), jnp.float32),
        scratch_types=(
            [
                pltpu.VMEM((QUAD, K), jnp.int32),
                pltpu.VMEM((QUAD, K), jnp.int32),
            ]
            + [pltpu.VMEM((K, 2 * F), jnp.float32)] * (4 * QUAD)
            + [pltpu.VMEM_SHARED((N, 2 * F), jnp.float32)]
            + [pltpu.SemaphoreType.DMA] * (4 * QUAD)
        ),
    )(_sc_body)


def _sc_layer(td, ts, c, src, dst, zero):
    return _get_sc_layer()(td, ts, c, src, dst, zero)


def kernel(x, edge_index, edge_attr, batch, W_emb, b_emb,
           Wf0, bf0, Ws0, bs0, g0, be0,
           Wf1, bf1, Ws1, bs1, g1, be1,
           Wf2, bf2, Ws2, bs2, g2, be2):
    src4 = edge_index[0].reshape(E // (QUAD * K), QUAD, K)
    dst4 = edge_index[1].reshape(E // (QUAD * K), QUAD, K)
    zero = jnp.zeros((N, 2 * F), jnp.float32)

    h = _embed(x, W_emb, b_emb.reshape(1, F))

    for Wf, bf, Ws, bs, g, be in (
        (Wf0, bf0, Ws0, bs0, g0, be0),
        (Wf1, bf1, Ws1, bs1, g1, be1),
        (Wf2, bf2, Ws2, bs2, g2, be2),
    ):
        wd = jnp.concatenate([Wf[:F], Ws[:F]], axis=1)
        wsrc = jnp.concatenate([Wf[F:2 * F], Ws[F:2 * F]], axis=1)
        wc = jnp.concatenate([Wf[2 * F:], Ws[2 * F:]], axis=1)
        bc = jnp.concatenate([bf, bs]).reshape(1, 2 * F)
        c = _cmat(edge_attr, wc, bc)
        td, ts = _tables(h, wd, wsrc)
        p = _sc_layer(td, ts, c, src4, dst4, zero)
        h = _bn(p, h, g.reshape(1, F), be.reshape(1, F))
    return h


# speedup vs baseline: 1.7081x; 1.1674x over previous
import functools

import jax
import jax.numpy as jnp
from jax import lax
from jax.experimental import pallas as pl
from jax.experimental.pallas import tpu as pltpu
from jax.experimental.pallas import tpu_sc as plsc

N = 10000
E = 320000
IN_F = 128
F = 64
D = 16

NC = 2
NS = 16
NW = NC * NS
EW = E // NW
K = 40
GQ = 10
NSET = 2
NOUTER = EW // (GQ * K)


def _embed_body(x_ref, w_ref, b_ref, h_ref):
    h_ref[...] = (
        jnp.dot(x_ref[...], w_ref[...], preferred_element_type=jnp.float32)
        + b_ref[...]
    )


def _embed(x, w, b):
    return pl.pallas_call(
        _embed_body,
        out_shape=jax.ShapeDtypeStruct((N, F), jnp.float32),
    )(x, w, b)


def _tables_body(h_ref, wd_ref, ws_ref, td_ref, ts_ref):
    h = h_ref[...]
    td_ref[...] = jnp.dot(h, wd_ref[...], preferred_element_type=jnp.float32)
    ts_ref[...] = jnp.dot(h, ws_ref[...], preferred_element_type=jnp.float32)


def _tables(h, wd, wsrc):
    return pl.pallas_call(
        _tables_body,
        out_shape=(
            jax.ShapeDtypeStruct((N, 2 * F), jnp.float32),
            jax.ShapeDtypeStruct((N, 2 * F), jnp.float32),
        ),
    )(h, wd, wsrc)


_EB = 16000


def _cmat_body(ea_ref, wc_ref, bc_ref, c_ref):
    c_ref[...] = (
        jnp.dot(ea_ref[...], wc_ref[...], preferred_element_type=jnp.float32)
        + bc_ref[...]
    )


def _cmat(ea, wc, bc):
    return pl.pallas_call(
        _cmat_body,
        grid=(E // _EB,),
        in_specs=[
            pl.BlockSpec((_EB, D), lambda i: (i, 0)),
            pl.BlockSpec((D, 2 * F), lambda i: (0, 0)),
            pl.BlockSpec((1, 2 * F), lambda i: (0, 0)),
        ],
        out_specs=pl.BlockSpec((_EB, 2 * F), lambda i: (i, 0)),
        out_shape=jax.ShapeDtypeStruct((E, 2 * F), jnp.float32),
    )(ea, wc, bc)


def _bn_body(p_ref, h_ref, g_ref, be_ref, o_ref):
    agg = p_ref[0, :, :F] + p_ref[1, :, :F]
    mu = jnp.mean(agg, axis=0, keepdims=True)
    cen = agg - mu
    var = jnp.mean(cen * cen, axis=0, keepdims=True)
    o_ref[...] = g_ref[...] * cen * lax.rsqrt(var + 1e-5) + be_ref[...] + h_ref[...]


def _bn(p, h, g, be):
    return pl.pallas_call(
        _bn_body,
        out_shape=jax.ShapeDtypeStruct((N, F), jnp.float32),
    )(p, h, g, be)


def _softplus16(z):
    u = jnp.exp(-jnp.abs(z))
    t = u / (2.0 + u)
    t2 = t * t
    y0 = t * (2.0 + t2 * (2.0 / 3.0 + t2 * 0.4))
    return jnp.maximum(z, 0.0) + y0


def _sc_body(td_hbm, ts_hbm, c_hbm, srcg_hbm, dstg_hbm, zero_hbm, out_hbm,
             idx_sg, idx_dg, *rest):
    bufs = rest[:4 * NSET]
    agg = rest[4 * NSET]
    sems = rest[4 * NSET + 1:]
    cid = lax.axis_index("c")
    sid = lax.axis_index("s")
    wid = sid * NC + cid
    wbase = pl.multiple_of(wid * EW, 8)

    @pl.when(sid == 0)
    def _():
        pltpu.sync_copy(zero_hbm, agg)

    for i in range(NSET):
        mb = bufs[4 * i + 3]

        @plsc.parallel_loop(0, K)
        def _(e):
            for j in range(F // 16):
                mb[e, pl.ds(F + 16 * j, 16)] = jnp.zeros((16,), jnp.float32)

    plsc.subcore_barrier()

    def compute(bd, bs, bc, mb):
        @plsc.parallel_loop(0, K, unroll=2)
        def _(e):
            for j in range(F // 16):
                f_sl = pl.ds(16 * j, 16)
                s_sl = pl.ds(F + 16 * j, 16)
                zf = bd[e, f_sl] + bs[e, f_sl] + bc[e, f_sl]
                zs = bd[e, s_sl] + bs[e, s_sl] + bc[e, s_sl]
                mb[e, f_sl] = _softplus16(zs) / (1.0 + jnp.exp(-zf))

    def outer(t, carry):
        pr = wid * NOUTER + t
        pltpu.sync_copy(srcg_hbm.at[pr], idx_sg)
        pltpu.sync_copy(dstg_hbm.at[pr], idx_dg)

        def start(q):
            s = q % NSET
            bd, bs, bc, _ = bufs[4 * s:4 * s + 4]
            semd, semsrc, semc, _ = sems[4 * s:4 * s + 4]
            return (
                pltpu.async_copy(td_hbm.at[idx_dg.at[q]], bd, semd),
                pltpu.async_copy(ts_hbm.at[idx_sg.at[q]], bs, semsrc),
                pltpu.async_copy(
                    c_hbm.at[pl.ds(wbase + (GQ * t + q) * K, K), :], bc, semc),
            )

        gh = {0: start(0), 1: start(1)}
        sh = {}
        for q in range(GQ):
            s = q % NSET
            bd, bs, bc, mb = bufs[4 * s:4 * s + 4]
            for h in gh.pop(q):
                h.wait()
            if q - NSET in sh:
                sh.pop(q - NSET).wait()
            compute(bd, bs, bc, mb)
            sh[q] = pltpu.async_copy(
                mb, agg.at[idx_dg.at[q]], sems[4 * s + 3], add=True)
            if q + NSET < GQ:
                gh[q + NSET] = start(q + NSET)
        for h in sh.values():
            h.wait()
        return carry

    lax.fori_loop(0, NOUTER, outer, 0)

    plsc.subcore_barrier()

    rows = 624
    sl = pl.ds(sid * rows, rows)
    pltpu.sync_copy(agg.at[sl, :], out_hbm.at[cid, sl, :])

    @pl.when(sid == 0)
    def _():
        tail = pl.ds(NS * rows, N - NS * rows)
        pltpu.sync_copy(agg.at[tail, :], out_hbm.at[cid, tail, :])


@functools.cache
def _get_sc_layer():
    return functools.partial(
        pl.kernel,
        mesh=plsc.VectorSubcoreMesh(core_axis_name="c", subcore_axis_name="s"),
        out_type=jax.ShapeDtypeStruct((NC, N, 2 * F), jnp.float32),
        scratch_types=(
            [
                pltpu.VMEM((GQ, K), jnp.int32),
                pltpu.VMEM((GQ, K), jnp.int32),
            ]
            + [pltpu.VMEM((K, 2 * F), jnp.float32)] * (4 * NSET)
            + [pltpu.VMEM_SHARED((N, 2 * F), jnp.float32)]
            + [pltpu.SemaphoreType.DMA] * (4 * NSET)
        ),
    )(_sc_body)


def _sc_layer(td, ts, c, src, dst, zero):
    return _get_sc_layer()(td, ts, c, src, dst, zero)


def kernel(x, edge_index, edge_attr, batch, W_emb, b_emb,
           Wf0, bf0, Ws0, bs0, g0, be0,
           Wf1, bf1, Ws1, bs1, g1, be1,
           Wf2, bf2, Ws2, bs2, g2, be2):
    srcg = edge_index[0].reshape(E // (GQ * K), GQ, K)
    dstg = edge_index[1].reshape(E // (GQ * K), GQ, K)
    zero = jnp.zeros((N, 2 * F), jnp.float32)

    h = _embed(x, W_emb, b_emb.reshape(1, F))

    for Wf, bf, Ws, bs, g, be in (
        (Wf0, bf0, Ws0, bs0, g0, be0),
        (Wf1, bf1, Ws1, bs1, g1, be1),
        (Wf2, bf2, Ws2, bs2, g2, be2),
    ):
        wd = jnp.concatenate([Wf[:F], Ws[:F]], axis=1)
        wsrc = jnp.concatenate([Wf[F:2 * F], Ws[F:2 * F]], axis=1)
        wc = jnp.concatenate([Wf[2 * F:], Ws[2 * F:]], axis=1)
        bc = jnp.concatenate([bf, bs]).reshape(1, 2 * F)
        c = _cmat(edge_attr, wc, bc)
        td, ts = _tables(h, wd, wsrc)
        p = _sc_layer(td, ts, c, srcg, dstg, zero)
        h = _bn(p, h, g.reshape(1, F), be.reshape(1, F))
    return h
